# Initial kernel scaffold; baseline (speedup 1.0000x reference)
import jax
import jax.numpy as jnp
from jax.experimental import pallas as pl


def kernel(x, edge_index, edge_pairs, W_in, b_in, g_in, be_in, Wl0, bl0, Wr0, gn0, bn0, Wl1, bl1, Wr1, gn1, bn1, Wl2, bl2, Wr2, gn2, bn2, Wd0, bd0, gd0, bed0, Wd1, bd1, gd1, bed1, Wd2, bd2):
    raise NotImplementedError("write your pallas kernel here")


# speedup vs baseline: 3.6886x; 3.6886x over previous
import functools

import jax
import jax.numpy as jnp
from jax import lax
from jax.experimental import pallas as pl
from jax.experimental.pallas import tpu as pltpu
from jax.experimental.pallas import tpu_sc as plsc

N = 10000
E = 320000
P = 100000
IN_DIM = 128
D = 256
H = 128
NC = 2
NS = 16
CH = 128
NP = 10240
RT = NP // NS
PP = 100096
EC = E // CH
PC = PP // CH
EPS = 1e-5
BN = 1000

@functools.lru_cache(maxsize=None)
def _mesh():
    return plsc.VectorSubcoreMesh(core_axis_name="c", subcore_axis_name="s",
                                  num_cores=NC, num_subcores=NS)


def _ln(t, g, b):
    mu = jnp.mean(t, axis=-1, keepdims=True)
    var = jnp.mean((t - mu) ** 2, axis=-1, keepdims=True)
    return (t - mu) * lax.rsqrt(var + EPS) * g + b


def _enc_body(x_ref, w_ref, b_ref, g_ref, be_ref, o_ref):
    t = jnp.dot(x_ref[...], w_ref[...], preferred_element_type=jnp.float32)
    t = jnp.maximum(_ln(t + b_ref[...], g_ref[...], be_ref[...]), 0.0)
    o_ref[0] = t[:, :H]
    o_ref[1] = t[:, H:]


def _encode(x, W_in, b_in, g_in, be_in):
    return pl.pallas_call(
        _enc_body,
        grid=(N // BN,),
        in_specs=[
            pl.BlockSpec((BN, IN_DIM), lambda i: (i, 0)),
            pl.BlockSpec((IN_DIM, D), lambda i: (0, 0)),
            pl.BlockSpec((1, D), lambda i: (0, 0)),
            pl.BlockSpec((1, D), lambda i: (0, 0)),
            pl.BlockSpec((1, D), lambda i: (0, 0)),
        ],
        out_specs=pl.BlockSpec((2, BN, H), lambda i: (0, i, 0)),
        out_shape=jax.ShapeDtypeStruct((2, N, H), jnp.float32),
    )(x, W_in, b_in.reshape(1, D), g_in.reshape(1, D), be_in.reshape(1, D))


def _conv_body(h_ref, a_ref, c_ref, wl_ref, bl_ref, wr_ref, g_ref, b_ref,
               o_ref, *, residual):
    h = jnp.concatenate([h_ref[0], h_ref[1]], axis=1)
    agg = jnp.concatenate([a_ref[0], a_ref[1]], axis=1)
    cnt = jnp.maximum(c_ref[0, :, 0:1] + c_ref[1, :, 0:1], 1.0)
    t = (jnp.dot(agg / cnt, wl_ref[...], preferred_element_type=jnp.float32)
         + bl_ref[...]
         + jnp.dot(h, wr_ref[...], preferred_element_type=jnp.float32))
    t = jnp.maximum(_ln(t, g_ref[...], b_ref[...]), 0.0)
    if residual:
        t = t + h
    o_ref[0] = t[:, :H]
    o_ref[1] = t[:, H:]


def _conv(h2, agg2, cnt2, Wl, bl, Wr, gn, bn, residual):
    return pl.pallas_call(
        functools.partial(_conv_body, residual=residual),
        grid=(N // BN,),
        in_specs=[
            pl.BlockSpec((2, BN, H), lambda i: (0, i, 0)),
            pl.BlockSpec((2, BN, H), lambda i: (0, i, 0)),
            pl.BlockSpec((2, BN, H), lambda i: (0, i, 0)),
            pl.BlockSpec((D, D), lambda i: (0, 0)),
            pl.BlockSpec((1, D), lambda i: (0, 0)),
            pl.BlockSpec((D, D), lambda i: (0, 0)),
            pl.BlockSpec((1, D), lambda i: (0, 0)),
            pl.BlockSpec((1, D), lambda i: (0, 0)),
        ],
        out_specs=pl.BlockSpec((2, BN, H), lambda i: (0, i, 0)),
        out_shape=jax.ShapeDtypeStruct((2, N, H), jnp.float32),
    )(h2, agg2, cnt2, Wl, bl.reshape(1, D), Wr, gn.reshape(1, D),
      bn.reshape(1, D))


def _dec_body(u_ref, v_ref, w0_ref, b0_ref, g0_ref, be0_ref,
              w1_ref, b1_ref, g1_ref, be1_ref, w2_ref, b2_ref, o_ref):
    zu = jnp.concatenate([u_ref[0], u_ref[1]], axis=1)
    zv = jnp.concatenate([v_ref[0], v_ref[1]], axis=1)
    t = jnp.dot(zu * zv, w0_ref[...], preferred_element_type=jnp.float32)
    t = jnp.maximum(_ln(t + b0_ref[...], g0_ref[...], be0_ref[...]), 0.0)
    t = jnp.dot(t, w1_ref[...], preferred_element_type=jnp.float32)
    t = jnp.maximum(_ln(t + b1_ref[...], g1_ref[...], be1_ref[...]), 0.0)
    o_ref[...] = (jnp.dot(t, w2_ref[...], preferred_element_type=jnp.float32)
                  + b2_ref[...])


def _decode(zu2, zv2, Wd0, bd0, gd0, bed0, Wd1, bd1, gd1, bed1, Wd2, bd2):
    return pl.pallas_call(
        _dec_body,
        grid=(P // BN,),
        in_specs=[
            pl.BlockSpec((2, BN, H), lambda i: (0, i, 0)),
            pl.BlockSpec((2, BN, H), lambda i: (0, i, 0)),
            pl.BlockSpec((D, D), lambda i: (0, 0)),
            pl.BlockSpec((1, D), lambda i: (0, 0)),
            pl.BlockSpec((1, D), lambda i: (0, 0)),
            pl.BlockSpec((1, D), lambda i: (0, 0)),
            pl.BlockSpec((D, D), lambda i: (0, 0)),
            pl.BlockSpec((1, D), lambda i: (0, 0)),
            pl.BlockSpec((1, D), lambda i: (0, 0)),
            pl.BlockSpec((1, D), lambda i: (0, 0)),
            pl.BlockSpec((D, 1), lambda i: (0, 0)),
            pl.BlockSpec((1, 1), lambda i: (0, 0)),
        ],
        out_specs=pl.BlockSpec((BN, 1), lambda i: (i, 0)),
        out_shape=jax.ShapeDtypeStruct((P, 1), jnp.float32),
    )(zu2, zv2, Wd0, bd0.reshape(1, D), gd0.reshape(1, D), bed0.reshape(1, D),
      Wd1, bd1.reshape(1, D), gd1.reshape(1, D), bed1.reshape(1, D),
      Wd2, bd2.reshape(1, 1))


def _fill_rows(buf, nrows, width, value):
    def body(i, carry):
        for j in range(width // 16):
            buf[i, pl.ds(j * 16, 16)] = jnp.full((16,), value, jnp.float32)
        return carry
    lax.fori_loop(0, nrows, body, 0)


def _agg_body(h2_hbm, src2_hbm, dst_hbm, out_hbm, idx_s, idx_d, rows,
              acc_sh, sem):
    c = lax.axis_index("c")
    s = lax.axis_index("s")
    _fill_rows(rows, CH, H, 0.0)
    for k in range(RT // CH):
        pltpu.sync_copy(rows, acc_sh.at[pl.ds(s * RT + k * CH, CH)])
    plsc.subcore_barrier()

    def step(i, carry):
        ch = i * NS + s

        @pl.when(ch < EC)
        def _():
            pltpu.sync_copy(src2_hbm.at[c, pl.ds(ch * CH, CH)], idx_s)
            pltpu.sync_copy(dst_hbm.at[pl.ds(ch * CH, CH)], idx_d)
            pltpu.async_copy(h2_hbm.at[idx_s], rows, sem).wait()
            pltpu.sync_copy(rows, acc_sh.at[idx_d], add=True)
        return carry

    lax.fori_loop(0, (EC + NS - 1) // NS, step, 0)
    plsc.subcore_barrier()
    pltpu.sync_copy(acc_sh.at[pl.ds(s * RT, RT)],
                    out_hbm.at[pl.ds(c * NP + s * RT, RT)])


@functools.lru_cache(maxsize=None)
def _sc_agg():
    return pl.kernel(
        _agg_body,
        out_type=jax.ShapeDtypeStruct((2 * NP, H), jnp.float32),
        mesh=_mesh(),
        scratch_types=[
            pltpu.VMEM((CH,), jnp.int32),
            pltpu.VMEM((CH,), jnp.int32),
            pltpu.VMEM((CH, H), jnp.float32),
            pltpu.VMEM_SHARED((NP, H), jnp.float32),
            pltpu.SemaphoreType.DMA,
        ],
    )


def _cnt_body(dst_hbm, out_hbm, idx_d, ones_v, cnt_sh):
    c = lax.axis_index("c")
    s = lax.axis_index("s")
    wid = s * NC + c
    _fill_rows(ones_v, CH, H, 0.0)
    for k in range(RT // CH):
        pltpu.sync_copy(ones_v, cnt_sh.at[pl.ds(s * RT + k * CH, CH)])
    _fill_rows(ones_v, CH, H, 1.0)
    plsc.subcore_barrier()

    def step(i, carry):
        ch = i * (NC * NS) + wid

        @pl.when(ch < EC)
        def _():
            pltpu.sync_copy(dst_hbm.at[pl.ds(ch * CH, CH)], idx_d)
            pltpu.sync_copy(ones_v, cnt_sh.at[idx_d], add=True)
        return carry

    lax.fori_loop(0, (EC + NC * NS - 1) // (NC * NS), step, 0)
    plsc.subcore_barrier()
    pltpu.sync_copy(cnt_sh.at[pl.ds(s * RT, RT)],
                    out_hbm.at[pl.ds(c * NP + s * RT, RT)])


@functools.lru_cache(maxsize=None)
def _sc_counts():
    return pl.kernel(
        _cnt_body,
        out_type=jax.ShapeDtypeStruct((2 * NP, H), jnp.float32),
        mesh=_mesh(),
        scratch_types=[
            pltpu.VMEM((CH,), jnp.int32),
            pltpu.VMEM((CH, H), jnp.float32),
            pltpu.VMEM_SHARED((NP, H), jnp.float32),
        ],
    )


def _pair_body(z2_hbm, u2_hbm, v2_hbm, zu_hbm, zv_hbm, idx_u, idx_v,
               rows_u, rows_v, sem_u, sem_v):
    c = lax.axis_index("c")
    s = lax.axis_index("s")

    def step(i, carry):
        ch = i * NS + s

        @pl.when(ch < PC)
        def _():
            pltpu.sync_copy(u2_hbm.at[c, pl.ds(ch * CH, CH)], idx_u)
            pltpu.sync_copy(v2_hbm.at[c, pl.ds(ch * CH, CH)], idx_v)
            cu = pltpu.async_copy(z2_hbm.at[idx_u], rows_u, sem_u)
            cv = pltpu.async_copy(z2_hbm.at[idx_v], rows_v, sem_v)
            cu.wait()
            cv.wait()
            pltpu.sync_copy(rows_u, zu_hbm.at[pl.ds(c * PP + ch * CH, CH)])
            pltpu.sync_copy(rows_v, zv_hbm.at[pl.ds(c * PP + ch * CH, CH)])
        return carry

    lax.fori_loop(0, (PC + NS - 1) // NS, step, 0)


@functools.lru_cache(maxsize=None)
def _sc_pair():
    return pl.kernel(
        _pair_body,
        out_type=(jax.ShapeDtypeStruct((2 * PP, H), jnp.float32),
                  jax.ShapeDtypeStruct((2 * PP, H), jnp.float32)),
        mesh=_mesh(),
        scratch_types=[
            pltpu.VMEM((CH,), jnp.int32),
            pltpu.VMEM((CH,), jnp.int32),
            pltpu.VMEM((CH, H), jnp.float32),
            pltpu.VMEM((CH, H), jnp.float32),
            pltpu.SemaphoreType.DMA,
            pltpu.SemaphoreType.DMA,
        ],
    )


def kernel(x, edge_index, edge_pairs, W_in, b_in, g_in, be_in,
           Wl0, bl0, Wr0, gn0, bn0,
           Wl1, bl1, Wr1, gn1, bn1,
           Wl2, bl2, Wr2, gn2, bn2,
           Wd0, bd0, gd0, bed0, Wd1, bd1, gd1, bed1, Wd2, bd2):
    src = edge_index[0]
    dst = edge_index[1]
    src2 = jnp.stack([src, src + N])
    u = edge_pairs[:, 0]
    v = edge_pairs[:, 1]
    zpad = jnp.zeros((PP - P,), jnp.int32)
    upad = jnp.concatenate([u, zpad])
    vpad = jnp.concatenate([v, zpad])
    u2 = jnp.stack([upad, upad + N])
    v2 = jnp.stack([vpad, vpad + N])

    cnt2 = _sc_counts()(dst).reshape(2, NP, H)
    h2 = _encode(x, W_in, b_in, g_in, be_in)

    convs = [(Wl0, bl0, Wr0, gn0, bn0),
             (Wl1, bl1, Wr1, gn1, bn1),
             (Wl2, bl2, Wr2, gn2, bn2)]
    for i, (Wl, bl, Wr, gn, bn) in enumerate(convs):
        agg = _sc_agg()(h2.reshape(2 * N, H), src2, dst)
        h2 = _conv(h2, agg.reshape(2, NP, H), cnt2, Wl, bl, Wr, gn, bn,
                   residual=(i > 0))

    zu, zv = _sc_pair()(h2.reshape(2 * N, H), u2, v2)
    out = _decode(zu.reshape(2, PP, H), zv.reshape(2, PP, H),
                  Wd0, bd0, gd0, bed0, Wd1, bd1, gd1, bed1, Wd2, bd2)
    return out[:, 0]
